# Initial kernel scaffold; baseline (speedup 1.0000x reference)
import jax
import jax.numpy as jnp
from jax.experimental import pallas as pl


def kernel(x):
    raise NotImplementedError("write your pallas kernel here")


# speedup vs baseline: 18.8598x; 18.8598x over previous
import functools

import jax
import jax.numpy as jnp
from jax import lax
from jax.experimental import pallas as pl
from jax.experimental.pallas import tpu as pltpu
from jax.experimental.pallas import tpu_sc as plsc

NC, NS, L = 2, 16, 16
NW = NC * NS

ROWS, COLS = 2048, 16384
N = ROWS * COLS
KTOT = 32 * ROWS

NPW = N // NW
CHUNK = 32768
NCHUNK = NPW // CHUNK

NB1 = 256
NB2 = 4096
NB3 = 4096

_mesh = plsc.VectorSubcoreMesh(
    core_axis_name="c", subcore_axis_name="s", num_cores=NC, num_subcores=NS
)


def _zero_i32(ref, n):
    z = jnp.zeros((16,), jnp.int32)

    def body(i, _):
        ref[pl.ds(i * 16, 16)] = z
        return 0

    lax.fori_loop(0, n // 16, body, 0)


def _select_hist(comb, nb, k_need):
    ngroups = nb // 16

    def gbody(i, carry):
        acc, g_sel, acc_at = carry
        g = ngroups - 1 - i
        v = comb[pl.ds(g * 16, 16)]
        s = jnp.sum(v)
        hit = jnp.logical_and(acc < k_need, acc + s >= k_need)
        g_sel = jnp.where(hit, g, g_sel)
        acc_at = jnp.where(hit, acc, acc_at)
        return (acc + s, g_sel, acc_at)

    _, g_sel, acc_at = lax.fori_loop(
        0, ngroups, gbody, (jnp.int32(0), jnp.int32(0), jnp.int32(0))
    )
    v = comb[pl.ds(g_sel * 16, 16)]
    inc = plsc.cumsum(v)
    total = jnp.sum(v)
    above = acc_at + (total - inc)
    hit = jnp.logical_and(above < k_need, above + v >= k_need)
    lane = lax.iota(jnp.int32, 16)
    b_sel = g_sel * 16 + jnp.sum(jnp.where(hit, lane, 0))
    k_at = k_need - jnp.sum(jnp.where(hit, above, 0))
    return b_sel, k_at


def _combine_workers(h_hbm, row, comb, nb):
    _zero_i32(comb, nb)

    def wbody(w, _):
        pltpu.sync_copy(h_hbm.at[w], row)

        def gbody(g, _):
            base = g * 16
            comb[pl.ds(base, 16)] = comb[pl.ds(base, 16)] + row[pl.ds(base, 16)]
            return 0

        lax.fori_loop(0, nb // 16, gbody, 0)
        return 0

    lax.fori_loop(0, NW, wbody, 0)


def _combine_lanes(hist, nb, outb):
    def body(g, _):
        base = g * 16
        acc = hist[pl.ds(base, 16)]
        for l in range(1, L):
            acc = acc + hist[pl.ds(l * nb + base, 16)]
        outb[pl.ds(base, 16)] = acc
        return 0

    lax.fori_loop(0, nb // 16, body, 0)


def _hist_data_pass(x_hbm, buf, hist, wid, nb, bin_shift, bin_mask,
                    prefix_shift, prefix_val):
    lane_base = lax.iota(jnp.int32, 16) * nb
    ones = jnp.ones((16,), jnp.int32)

    def chunk_body(c, _):
        off = pl.multiple_of(wid * NPW + c * CHUNK, CHUNK)
        pltpu.sync_copy(x_hbm.at[pl.ds(off, CHUNK)], buf)

        def vbody(i, _):
            v = buf[pl.ds(i * 16, 16)]
            bits = lax.bitcast_convert_type(jnp.maximum(v, 0.0), jnp.int32)
            bin_ = jnp.bitwise_and(
                lax.shift_right_logical(bits, bin_shift), bin_mask
            )
            if prefix_shift is None:
                mask = None
            else:
                mask = lax.shift_right_logical(bits, prefix_shift) == prefix_val
            plsc.addupdate_scatter(hist, [lane_base + bin_], ones, mask=mask)
            return 0

        lax.fori_loop(0, CHUNK // 16, vbody, 0)
        return 0

    lax.fori_loop(0, NCHUNK, chunk_body, 0)


@functools.partial(
    pl.kernel,
    out_type=jax.ShapeDtypeStruct((NW, NB1), jnp.int32),
    mesh=_mesh,
    compiler_params=pltpu.CompilerParams(needs_layout_passes=False),
    scratch_types=[
        pltpu.VMEM((CHUNK,), jnp.float32),
        pltpu.VMEM((L * NB1,), jnp.int32),
        pltpu.VMEM((NB1,), jnp.int32),
    ],
)
def _hist1_kernel(x_hbm, out_hbm, buf, hist, outb):
    wid = lax.axis_index("s") * NC + lax.axis_index("c")
    _zero_i32(hist, L * NB1)
    _hist_data_pass(x_hbm, buf, hist, wid, NB1, 24, NB1 - 1, None, None)
    _combine_lanes(hist, NB1, outb)
    pltpu.sync_copy(outb, out_hbm.at[wid])


@functools.partial(
    pl.kernel,
    out_type=jax.ShapeDtypeStruct((NW, NB2), jnp.int32),
    mesh=_mesh,
    compiler_params=pltpu.CompilerParams(needs_layout_passes=False),
    scratch_types=[
        pltpu.VMEM((CHUNK,), jnp.float32),
        pltpu.VMEM((L * NB2,), jnp.int32),
        pltpu.VMEM((NB2,), jnp.int32),
        pltpu.VMEM((NB1,), jnp.int32),
        pltpu.VMEM((NB1,), jnp.int32),
    ],
)
def _hist2_kernel(x_hbm, h1_hbm, out_hbm, buf, hist, outb, row1, comb1):
    wid = lax.axis_index("s") * NC + lax.axis_index("c")
    _combine_workers(h1_hbm, row1, comb1, NB1)
    b1, _ = _select_hist(comb1, NB1, KTOT)
    _zero_i32(hist, L * NB2)
    _hist_data_pass(x_hbm, buf, hist, wid, NB2, 12, NB2 - 1, 24, b1)
    _combine_lanes(hist, NB2, outb)
    pltpu.sync_copy(outb, out_hbm.at[wid])


@functools.partial(
    pl.kernel,
    out_type=jax.ShapeDtypeStruct((NW, NB3), jnp.int32),
    mesh=_mesh,
    compiler_params=pltpu.CompilerParams(needs_layout_passes=False),
    scratch_types=[
        pltpu.VMEM((CHUNK,), jnp.float32),
        pltpu.VMEM((L * NB3,), jnp.int32),
        pltpu.VMEM((NB3,), jnp.int32),
        pltpu.VMEM((NB1,), jnp.int32),
        pltpu.VMEM((NB1,), jnp.int32),
        pltpu.VMEM((NB2,), jnp.int32),
        pltpu.VMEM((NB2,), jnp.int32),
    ],
)
def _hist3_kernel(x_hbm, h1_hbm, h2_hbm, out_hbm, buf, hist, outb,
                  row1, comb1, row2, comb2):
    wid = lax.axis_index("s") * NC + lax.axis_index("c")
    _combine_workers(h1_hbm, row1, comb1, NB1)
    b1, k1 = _select_hist(comb1, NB1, KTOT)
    _combine_workers(h2_hbm, row2, comb2, NB2)
    b2, _ = _select_hist(comb2, NB2, k1)
    p2 = b1 * NB2 + b2
    _zero_i32(hist, L * NB3)
    _hist_data_pass(x_hbm, buf, hist, wid, NB3, 0, NB3 - 1, 12, p2)
    _combine_lanes(hist, NB3, outb)
    pltpu.sync_copy(outb, out_hbm.at[wid])


@functools.partial(
    pl.kernel,
    out_type=jax.ShapeDtypeStruct((16,), jnp.int32),
    mesh=_mesh,
    compiler_params=pltpu.CompilerParams(needs_layout_passes=False),
    scratch_types=[
        pltpu.VMEM((NB1,), jnp.int32),
        pltpu.VMEM((NB1,), jnp.int32),
        pltpu.VMEM((NB2,), jnp.int32),
        pltpu.VMEM((NB2,), jnp.int32),
        pltpu.VMEM((NB3,), jnp.int32),
        pltpu.VMEM((NB3,), jnp.int32),
        pltpu.VMEM((16,), jnp.int32),
    ],
)
def _thresh_kernel(h1_hbm, h2_hbm, h3_hbm, out_hbm, row1, comb1, row2, comb2,
                   row3, comb3, tout):
    wid = lax.axis_index("s") * NC + lax.axis_index("c")

    @pl.when(wid == 0)
    def _():
        _combine_workers(h1_hbm, row1, comb1, NB1)
        b1, k1 = _select_hist(comb1, NB1, KTOT)
        _combine_workers(h2_hbm, row2, comb2, NB2)
        b2, k2 = _select_hist(comb2, NB2, k1)
        _combine_workers(h3_hbm, row3, comb3, NB3)
        b3, _ = _select_hist(comb3, NB3, k2)
        t = b1 * (1 << 24) + b2 * (1 << 12) + b3
        tout[...] = jnp.zeros((16,), jnp.int32) + t
        pltpu.sync_copy(tout, out_hbm)


BLK_R = 128


def _finalize_body(t_ref, x_ref, o_ref):
    acts = jnp.maximum(x_ref[...], 0.0)
    bits = lax.bitcast_convert_type(acts, jnp.int32)
    o_ref[...] = jnp.where(bits >= t_ref[0], acts, 0.0)


def _finalize(t, x):
    return pl.pallas_call(
        _finalize_body,
        grid=(ROWS // BLK_R,),
        in_specs=[
            pl.BlockSpec(memory_space=pltpu.SMEM),
            pl.BlockSpec((BLK_R, COLS), lambda i: (i, 0)),
        ],
        out_specs=pl.BlockSpec((BLK_R, COLS), lambda i: (i, 0)),
        out_shape=jax.ShapeDtypeStruct((ROWS, COLS), jnp.float32),
    )(t, x)


def kernel(x):
    xf = x.reshape(-1)
    h1 = _hist1_kernel(xf)
    h2 = _hist2_kernel(xf, h1)
    h3 = _hist3_kernel(xf, h1, h2)
    t = _thresh_kernel(h1, h2, h3)
    return _finalize(t, x)
